# Initial kernel scaffold; baseline (speedup 1.0000x reference)
import jax
import jax.numpy as jnp
from jax.experimental import pallas as pl


def kernel(x, edge_index, affine_w, affine_b, lin_W, lin_b, ln_gamma, ln_beta, prelu_a):
    raise NotImplementedError("write your pallas kernel here")


# speedup vs baseline: 2.0775x; 2.0775x over previous
import functools

import jax
import jax.numpy as jnp
from jax import lax
from jax.experimental import pallas as pl
from jax.experimental.pallas import tpu as pltpu
from jax.experimental.pallas import tpu_sc as plsc

N_TILES = 32
LANES = 16


def _row_block(n):
    for rb in (2000, 1000, 500, 250, 200, 125, 100, 50, 25, 10, 8, 5, 4, 2, 1):
        if n % rb == 0 and rb % 8 == 0 or n % rb == 0 and rb < 8:
            return rb
    return 1


def _pre_body(x_ref, w1t_ref, w2t_ref, t_ref, u_ref):
    xb = x_ref[...]
    d = xb.shape[1]
    t_ref[:, :d] = xb
    t_ref[:, d:] = jnp.dot(xb, w2t_ref[...], preferred_element_type=jnp.float32)
    u_ref[...] = jnp.dot(xb, w1t_ref[...], preferred_element_type=jnp.float32)


def _tc_pre(x, w1t, w2t):
    n, d = x.shape
    rb = _row_block(n)
    return pl.pallas_call(
        _pre_body,
        grid=(n // rb,),
        in_specs=[
            pl.BlockSpec((rb, d), lambda i: (i, 0)),
            pl.BlockSpec((d, d), lambda i: (0, 0)),
            pl.BlockSpec((d, d), lambda i: (0, 0)),
        ],
        out_specs=[
            pl.BlockSpec((rb, 2 * d), lambda i: (i, 0)),
            pl.BlockSpec((rb, d), lambda i: (i, 0)),
        ],
        out_shape=[
            jax.ShapeDtypeStruct((n, 2 * d), jnp.float32),
            jax.ShapeDtypeStruct((n, d), jnp.float32),
        ],
    )(x, w1t, w2t)


def _make_sc(e_pad, n_pad, d, npt, chunk, grp):
    nb = d // LANES
    n_chunks = e_pad // chunk
    mesh = plsc.VectorSubcoreMesh(core_axis_name="c", subcore_axis_name="s")

    @functools.partial(
        pl.kernel,
        out_type=[
            jax.ShapeDtypeStruct((n_pad, d), jnp.float32),
            jax.ShapeDtypeStruct((N_TILES, 8, LANES), jnp.float32),
        ],
        mesh=mesh,
        compiler_params=pltpu.CompilerParams(needs_layout_passes=False),
        scratch_types=[
            pltpu.VMEM((chunk,), jnp.int32),
            pltpu.VMEM((chunk,), jnp.int32),
            pltpu.VMEM((chunk,), jnp.int32),
            pltpu.VMEM((chunk + LANES,), jnp.int32),
            pltpu.VMEM((npt, d), jnp.float32),
            pltpu.VMEM((npt, d), jnp.float32),
            pltpu.VMEM((grp, 2 * d), jnp.float32),
            pltpu.VMEM((8, LANES), jnp.float32),
            pltpu.SemaphoreType.DMA,
        ],
    )
    def sc_kernel(src_hbm, dst_hbm, t_hbm, xpad_hbm, m_out, part_out,
                  sbuf, dbuf, slist, dlist, xloc, accM, gbuf, stage, sem):
        wid = lax.axis_index("s") * 2 + lax.axis_index("c")
        base = wid * npt

        neg_inf = jnp.full((LANES,), -jnp.inf, dtype=jnp.float32)
        zerof = jnp.zeros((LANES,), jnp.float32)
        zeroi = jnp.zeros((LANES,), jnp.int32)

        def init_row(r, carry):
            for j in range(nb):
                accM[r, pl.ds(j * LANES, LANES)] = neg_inf
            return carry

        lax.fori_loop(0, npt, init_row, 0)

        def init_sl(i, carry):
            slist[pl.ds(i * LANES, LANES)] = zeroi
            return carry

        lax.fori_loop(0, chunk // LANES, init_sl, 0)

        pltpu.sync_copy(xpad_hbm.at[pl.ds(base, npt)], xloc)

        z8 = tuple(jnp.zeros((LANES,), jnp.float32) for _ in range(nb))

        def chunk_body(ci, carry):
            pltpu.sync_copy(src_hbm.at[pl.ds(ci * chunk, chunk)], sbuf)
            pltpu.sync_copy(dst_hbm.at[pl.ds(ci * chunk, chunk)], dbuf)

            def filt(i, pos):
                dv = dbuf[pl.ds(i * LANES, LANES)]
                sv = sbuf[pl.ds(i * LANES, LANES)]
                msk = (dv >= base) & (dv < base + npt)
                key = jnp.where(msk, 0, 1).astype(jnp.int32)
                _, s_srt = plsc.sort_key_val(key, sv)
                _, d_srt = plsc.sort_key_val(key, dv - base)
                slist[pl.ds(pos, LANES)] = s_srt
                dlist[pl.ds(pos, LANES)] = d_srt
                pc = plsc.all_reduce_population_count(msk)
                return pos + pc[0]

            m_c = lax.fori_loop(0, chunk // LANES, filt, jnp.int32(0))
            ng = (m_c + (grp - 1)) // grp

            def group_body(g, carry2):
                g0 = g * grp
                pltpu.async_copy(t_hbm.at[slist.at[pl.ds(g0, grp)]], gbuf,
                                 sem).wait()
                kn = jnp.minimum(grp, m_c - g0)

                def edge_body(k, carry3):
                    dacc, s2acc, crossacc = carry3
                    li = dlist[pl.ds(g0 + k, LANES)][0]
                    dn, qn, cn = [], [], []
                    for j in range(nb):
                        sl = pl.ds(j * LANES, LANES)
                        xp = gbuf[k, sl]
                        xd = xloc[li, sl]
                        dn.append(dacc[j] + (xp - xd))
                        qn.append(s2acc[j] + (xp * xp + xd * xd))
                        cn.append(crossacc[j] + xp * xd)
                        vp = gbuf[k, pl.ds(d + j * LANES, LANES)]
                        accM[li, sl] = jnp.maximum(accM[li, sl], vp)
                    return (tuple(dn), tuple(qn), tuple(cn))

                return lax.fori_loop(0, kn, edge_body, carry2)

            return lax.fori_loop(0, ng, group_body, carry)

        dacc, s2acc, crossacc = lax.fori_loop(0, n_chunks, chunk_body,
                                              (z8, z8, z8))

        def vsum(acc):
            t = acc[0]
            for j in range(1, nb):
                t = t + acc[j]
            return t

        stage[0, :] = vsum(dacc)
        stage[1, :] = vsum(s2acc)
        stage[2, :] = vsum(crossacc)
        for j in range(3, 8):
            stage[j, :] = zerof
        pltpu.sync_copy(stage, part_out.at[wid])
        pltpu.sync_copy(accM, m_out.at[pl.ds(base, npt)])

    return sc_kernel


def _fin_body(m_ref, u_ref, v_ref, part_ref, cvec_ref, gam_ref, bet_ref,
              pa_ref, o_ref, *, n_total):
    part = part_ref[...]
    s1 = jnp.sum(part[:, 0, :])
    sq = jnp.sum(part[:, 1, :])
    cross = jnp.sum(part[:, 2, :])
    s2 = sq - 2.0 * cross
    var = (s2 - s1 * s1 / n_total) / (n_total - 1.0)
    inv = 1.0 / (jnp.sqrt(var) + 1e-5)

    m = m_ref[...]
    agg = u_ref[...] + cvec_ref[...] + inv * (m - v_ref[...])
    agg = jnp.where(m == -jnp.inf, 0.0, agg)
    mu = jnp.mean(agg, axis=-1, keepdims=True)
    dev = agg - mu
    va = jnp.mean(dev * dev, axis=-1, keepdims=True)
    h = dev * lax.rsqrt(va + 1e-5)
    h = h * gam_ref[...] + bet_ref[...]
    o_ref[...] = jnp.where(h >= 0.0, h, pa_ref[0, 0] * h)


def _tc_fin(m, u, v, part, cvec, gam, bet, pa, n_total):
    n, d = u.shape
    rb = _row_block(n)
    nt = part.shape[0]
    return pl.pallas_call(
        functools.partial(_fin_body, n_total=float(n_total)),
        grid=(n // rb,),
        in_specs=[
            pl.BlockSpec((rb, d), lambda i: (i, 0)),
            pl.BlockSpec((rb, d), lambda i: (i, 0)),
            pl.BlockSpec((rb, d), lambda i: (i, 0)),
            pl.BlockSpec((nt, 8, LANES), lambda i: (0, 0, 0)),
            pl.BlockSpec((1, d), lambda i: (0, 0)),
            pl.BlockSpec((1, d), lambda i: (0, 0)),
            pl.BlockSpec((1, d), lambda i: (0, 0)),
            pl.BlockSpec((1, 1), lambda i: (0, 0)),
        ],
        out_specs=pl.BlockSpec((rb, d), lambda i: (i, 0)),
        out_shape=jax.ShapeDtypeStruct((n, d), jnp.float32),
    )(m, u, v, part, cvec, gam, bet, pa)


def kernel(x, edge_index, affine_w, affine_b, lin_W, lin_b, ln_gamma, ln_beta,
           prelu_a):
    n, d = x.shape
    e = edge_index.shape[1]
    src = edge_index[0].astype(jnp.int32)
    dst = edge_index[1].astype(jnp.int32)

    w1 = lin_W[:, :d]
    w2 = lin_W[:, d:]
    w1t = w1.T
    w2t = (w2 * affine_w[None, :]).T
    cvec = (affine_b @ w2.T + lin_b)[None, :]

    npt = (-(-n // N_TILES) + 7) // 8 * 8
    n_pad = N_TILES * npt
    chunk, grp = 2000, 32
    e_pad = -(-e // chunk) * chunk
    if e_pad != e:
        src = jnp.pad(src, (0, e_pad - e))
        dst = jnp.pad(dst, (0, e_pad - e), constant_values=jnp.int32(2 ** 30))
    xpad = jnp.pad(x, ((0, n_pad - n), (0, 0)))

    t_tab, u = _tc_pre(x, w1t, w2t)
    m_full, part = _make_sc(e_pad, n_pad, d, npt, chunk, grp)(
        src, dst, t_tab, xpad)
    out = _tc_fin(m_full[:n], u, t_tab[:, d:], part, cvec,
                  ln_gamma[None, :], ln_beta[None, :],
                  jnp.reshape(prelu_a, (1, 1)), e * d)
    return out


# speedup vs baseline: 2.3323x; 1.1227x over previous
import functools

import jax
import jax.numpy as jnp
from jax import lax
from jax.experimental import pallas as pl
from jax.experimental.pallas import tpu as pltpu
from jax.experimental.pallas import tpu_sc as plsc

N_TILES = 32
LANES = 16


def _row_block(n):
    for rb in (2000, 1000, 500, 250, 200, 125, 100, 50, 25, 10, 8, 5, 4, 2, 1):
        if n % rb == 0 and rb % 8 == 0 or n % rb == 0 and rb < 8:
            return rb
    return 1


def _pre_body(x_ref, w1t_ref, w2t_ref, t_ref, u_ref):
    xb = x_ref[...]
    d = xb.shape[1]
    t_ref[:, :d] = xb
    t_ref[:, d:] = jnp.dot(xb, w2t_ref[...], preferred_element_type=jnp.float32)
    u_ref[...] = jnp.dot(xb, w1t_ref[...], preferred_element_type=jnp.float32)


def _tc_pre(x, w1t, w2t):
    n, d = x.shape
    rb = _row_block(n)
    return pl.pallas_call(
        _pre_body,
        grid=(n // rb,),
        in_specs=[
            pl.BlockSpec((rb, d), lambda i: (i, 0)),
            pl.BlockSpec((d, d), lambda i: (0, 0)),
            pl.BlockSpec((d, d), lambda i: (0, 0)),
        ],
        out_specs=[
            pl.BlockSpec((rb, 2 * d), lambda i: (i, 0)),
            pl.BlockSpec((rb, d), lambda i: (i, 0)),
        ],
        out_shape=[
            jax.ShapeDtypeStruct((n, 2 * d), jnp.float32),
            jax.ShapeDtypeStruct((n, d), jnp.float32),
        ],
    )(x, w1t, w2t)


def _make_sc(e_pad, n_pad, d, npt, chunk, grp):
    nb = d // LANES
    n_chunks = e_pad // chunk
    mesh = plsc.VectorSubcoreMesh(core_axis_name="c", subcore_axis_name="s")

    @functools.partial(
        pl.kernel,
        out_type=[
            jax.ShapeDtypeStruct((n_pad, d), jnp.float32),
            jax.ShapeDtypeStruct((N_TILES, 8, LANES), jnp.float32),
        ],
        mesh=mesh,
        compiler_params=pltpu.CompilerParams(needs_layout_passes=False),
        scratch_types=[
            pltpu.VMEM((chunk,), jnp.int32),
            pltpu.VMEM((chunk,), jnp.int32),
            pltpu.VMEM((chunk,), jnp.int32),
            pltpu.VMEM((chunk,), jnp.int32),
            pltpu.VMEM((chunk,), jnp.int32),
            pltpu.VMEM((chunk + LANES,), jnp.int32),
            pltpu.VMEM((npt, d), jnp.float32),
            pltpu.VMEM((npt, d), jnp.float32),
            pltpu.VMEM((grp, 2 * d), jnp.float32),
            pltpu.VMEM((grp, 2 * d), jnp.float32),
            pltpu.VMEM((8, LANES), jnp.float32),
            pltpu.SemaphoreType.DMA,
            pltpu.SemaphoreType.DMA,
            pltpu.SemaphoreType.DMA,
            pltpu.SemaphoreType.DMA,
            pltpu.SemaphoreType.DMA,
        ],
    )
    def sc_kernel(src_hbm, dst_hbm, t_hbm, xpad_hbm, m_out, part_out,
                  sbuf0, sbuf1, dbuf0, dbuf1, slist, dlist, xloc, accM,
                  gbuf0, gbuf1, stage, sem_x, sem_c0, sem_c1, sem_g0, sem_g1):
        wid = lax.axis_index("s") * 2 + lax.axis_index("c")
        base = wid * npt
        sbuf = (sbuf0, sbuf1)
        dbuf = (dbuf0, dbuf1)
        gbuf = (gbuf0, gbuf1)
        sem_c = (sem_c0, sem_c1)
        sem_g = (sem_g0, sem_g1)

        neg_inf = jnp.full((LANES,), -jnp.inf, dtype=jnp.float32)
        zerof = jnp.zeros((LANES,), jnp.float32)
        zeroi = jnp.zeros((LANES,), jnp.int32)

        xcp = pltpu.async_copy(xpad_hbm.at[pl.ds(base, npt)], xloc, sem_x)
        pltpu.async_copy(src_hbm.at[pl.ds(0, chunk)], sbuf[0], sem_c[0])
        pltpu.async_copy(dst_hbm.at[pl.ds(0, chunk)], dbuf[0], sem_c[0])

        def init_row(r, carry):
            for j in range(nb):
                accM[r, pl.ds(j * LANES, LANES)] = neg_inf
            return carry

        lax.fori_loop(0, npt, init_row, 0)

        def init_sl(i, carry):
            slist[pl.ds(i * LANES, LANES)] = zeroi
            return carry

        lax.fori_loop(0, chunk // LANES, init_sl, 0)
        xcp.wait()

        z8 = tuple(jnp.zeros((LANES,), jnp.float32) for _ in range(nb))

        def chunk_pair(cp, carry):
            for b in range(2):
                ci = 2 * cp + b

                @pl.when(ci + 1 < n_chunks)
                def _():
                    nxt = pl.ds((ci + 1) * chunk, chunk)
                    pltpu.async_copy(src_hbm.at[nxt], sbuf[1 - b],
                                     sem_c[1 - b])
                    pltpu.async_copy(dst_hbm.at[nxt], dbuf[1 - b],
                                     sem_c[1 - b])

                cur = pl.ds(ci * chunk, chunk)
                pltpu.make_async_copy(src_hbm.at[cur], sbuf[b],
                                      sem_c[b]).wait()
                pltpu.make_async_copy(dst_hbm.at[cur], dbuf[b],
                                      sem_c[b]).wait()

                def filt(i, pos):
                    dv = dbuf[b][pl.ds(i * LANES, LANES)]
                    sv = sbuf[b][pl.ds(i * LANES, LANES)]
                    msk = (dv >= base) & (dv < base + npt)
                    pc = plsc.all_reduce_population_count(msk)

                    @pl.when(pc[0] > 0)
                    def _():
                        key = jnp.where(msk, 0, 1).astype(jnp.int32)
                        _, s_srt = plsc.sort_key_val(key, sv)
                        _, d_srt = plsc.sort_key_val(key, dv - base)
                        slist[pl.ds(pos, LANES)] = s_srt
                        dlist[pl.ds(pos, LANES)] = d_srt

                    return pos + pc[0]

                m_c = lax.fori_loop(0, chunk // LANES, filt, jnp.int32(0))
                ng = (m_c + (grp - 1)) // grp

                @pl.when(ng > 0)
                def _():
                    pltpu.async_copy(t_hbm.at[slist.at[pl.ds(0, grp)]],
                                     gbuf[0], sem_g[0])

                def group_pair(gp, carry2):
                    for gb in range(2):
                        g = 2 * gp + gb
                        g0 = g * grp

                        @pl.when(g + 1 < ng)
                        def _():
                            pltpu.async_copy(
                                t_hbm.at[slist.at[pl.ds(g0 + grp, grp)]],
                                gbuf[1 - gb], sem_g[1 - gb])

                        @pl.when(g < ng)
                        def _():
                            pltpu.make_async_copy(
                                t_hbm.at[slist.at[pl.ds(g0, grp)]],
                                gbuf[gb], sem_g[gb]).wait()

                        kn = jnp.clip(m_c - g0, 0, grp)

                        def edge_body(k, carry3):
                            dacc, s2acc, crossacc = carry3
                            li = dlist[pl.ds(g0 + k, LANES)][0]
                            dn, qn, cn = [], [], []
                            for j in range(nb):
                                sl = pl.ds(j * LANES, LANES)
                                xp = gbuf[gb][k, sl]
                                xd = xloc[li, sl]
                                dn.append(dacc[j] + (xp - xd))
                                qn.append(s2acc[j] + (xp * xp + xd * xd))
                                cn.append(crossacc[j] + xp * xd)
                                vp = gbuf[gb][k, pl.ds(d + j * LANES, LANES)]
                                accM[li, sl] = jnp.maximum(accM[li, sl], vp)
                            return (tuple(dn), tuple(qn), tuple(cn))

                        carry2 = lax.fori_loop(0, kn, edge_body, carry2)
                    return carry2

                carry = lax.fori_loop(0, (ng + 1) // 2, group_pair, carry)
            return carry

        dacc, s2acc, crossacc = lax.fori_loop(0, n_chunks // 2, chunk_pair,
                                              (z8, z8, z8))

        def vsum(acc):
            t = acc[0]
            for j in range(1, nb):
                t = t + acc[j]
            return t

        stage[0, :] = vsum(dacc)
        stage[1, :] = vsum(s2acc)
        stage[2, :] = vsum(crossacc)
        for j in range(3, 8):
            stage[j, :] = zerof
        pltpu.sync_copy(stage, part_out.at[wid])
        pltpu.sync_copy(accM, m_out.at[pl.ds(base, npt)])

    return sc_kernel


def _fin_body(m_ref, u_ref, v_ref, part_ref, cvec_ref, gam_ref, bet_ref,
              pa_ref, o_ref, *, n_total):
    part = part_ref[...]
    s1 = jnp.sum(part[:, 0, :])
    sq = jnp.sum(part[:, 1, :])
    cross = jnp.sum(part[:, 2, :])
    s2 = sq - 2.0 * cross
    var = (s2 - s1 * s1 / n_total) / (n_total - 1.0)
    inv = 1.0 / (jnp.sqrt(var) + 1e-5)

    m = m_ref[...]
    agg = u_ref[...] + cvec_ref[...] + inv * (m - v_ref[...])
    agg = jnp.where(m == -jnp.inf, 0.0, agg)
    mu = jnp.mean(agg, axis=-1, keepdims=True)
    dev = agg - mu
    va = jnp.mean(dev * dev, axis=-1, keepdims=True)
    h = dev * lax.rsqrt(va + 1e-5)
    h = h * gam_ref[...] + bet_ref[...]
    o_ref[...] = jnp.where(h >= 0.0, h, pa_ref[0, 0] * h)


def _tc_fin(m, u, v, part, cvec, gam, bet, pa, n_total):
    n, d = u.shape
    rb = _row_block(n)
    nt = part.shape[0]
    return pl.pallas_call(
        functools.partial(_fin_body, n_total=float(n_total)),
        grid=(n // rb,),
        in_specs=[
            pl.BlockSpec((rb, d), lambda i: (i, 0)),
            pl.BlockSpec((rb, d), lambda i: (i, 0)),
            pl.BlockSpec((rb, d), lambda i: (i, 0)),
            pl.BlockSpec((nt, 8, LANES), lambda i: (0, 0, 0)),
            pl.BlockSpec((1, d), lambda i: (0, 0)),
            pl.BlockSpec((1, d), lambda i: (0, 0)),
            pl.BlockSpec((1, d), lambda i: (0, 0)),
            pl.BlockSpec((1, 1), lambda i: (0, 0)),
        ],
        out_specs=pl.BlockSpec((rb, d), lambda i: (i, 0)),
        out_shape=jax.ShapeDtypeStruct((n, d), jnp.float32),
    )(m, u, v, part, cvec, gam, bet, pa)


def kernel(x, edge_index, affine_w, affine_b, lin_W, lin_b, ln_gamma, ln_beta,
           prelu_a):
    n, d = x.shape
    e = edge_index.shape[1]
    src = edge_index[0].astype(jnp.int32)
    dst = edge_index[1].astype(jnp.int32)

    w1 = lin_W[:, :d]
    w2 = lin_W[:, d:]
    w1t = w1.T
    w2t = (w2 * affine_w[None, :]).T
    cvec = (affine_b @ w2.T + lin_b)[None, :]

    npt = (-(-n // N_TILES) + 7) // 8 * 8
    n_pad = N_TILES * npt
    chunk, grp = 2000, 32
    e_pad = -(-e // (2 * chunk)) * (2 * chunk)
    if e_pad != e:
        src = jnp.pad(src, (0, e_pad - e))
        dst = jnp.pad(dst, (0, e_pad - e), constant_values=jnp.int32(2 ** 30))
    xpad = jnp.pad(x, ((0, n_pad - n), (0, 0)))

    t_tab, u = _tc_pre(x, w1t, w2t)
    m_full, part = _make_sc(e_pad, n_pad, d, npt, chunk, grp)(
        src, dst, t_tab, xpad)
    out = _tc_fin(m_full[:n], u, t_tab[:, d:], part, cvec,
                  ln_gamma[None, :], ln_beta[None, :],
                  jnp.reshape(prelu_a, (1, 1)), e * d)
    return out


# speedup vs baseline: 2.6330x; 1.1289x over previous
import functools

import jax
import jax.numpy as jnp
from jax import lax
from jax.experimental import pallas as pl
from jax.experimental.pallas import tpu as pltpu
from jax.experimental.pallas import tpu_sc as plsc

N_TILES = 32
LANES = 16


def _row_block(n):
    for rb in (2000, 1000, 500, 250, 200, 125, 100, 50, 25, 10, 8, 5, 4, 2, 1):
        if n % rb == 0 and rb % 8 == 0 or n % rb == 0 and rb < 8:
            return rb
    return 1


def _pre_body(x_ref, w1t_ref, w2t_ref, t_ref, u_ref):
    xb = x_ref[...]
    d = xb.shape[1]
    t_ref[:, :d] = xb
    t_ref[:, d:] = jnp.dot(xb, w2t_ref[...], preferred_element_type=jnp.float32)
    u_ref[...] = jnp.dot(xb, w1t_ref[...], preferred_element_type=jnp.float32)


def _tc_pre(x, w1t, w2t):
    n, d = x.shape
    rb = _row_block(n)
    return pl.pallas_call(
        _pre_body,
        grid=(n // rb,),
        in_specs=[
            pl.BlockSpec((rb, d), lambda i: (i, 0)),
            pl.BlockSpec((d, d), lambda i: (0, 0)),
            pl.BlockSpec((d, d), lambda i: (0, 0)),
        ],
        out_specs=[
            pl.BlockSpec((rb, 2 * d), lambda i: (i, 0)),
            pl.BlockSpec((rb, d), lambda i: (i, 0)),
        ],
        out_shape=[
            jax.ShapeDtypeStruct((n, 2 * d), jnp.float32),
            jax.ShapeDtypeStruct((n, d), jnp.float32),
        ],
    )(x, w1t, w2t)


def _make_sc(e_pad, n_pad, d, npt, chunk, grp):
    nb = d // LANES
    n_chunks = e_pad // chunk
    mesh = plsc.VectorSubcoreMesh(core_axis_name="c", subcore_axis_name="s")

    @functools.partial(
        pl.kernel,
        out_type=[
            jax.ShapeDtypeStruct((n_pad, d), jnp.float32),
            jax.ShapeDtypeStruct((N_TILES, 8, LANES), jnp.float32),
        ],
        mesh=mesh,
        compiler_params=pltpu.CompilerParams(needs_layout_passes=False),
        scratch_types=[
            pltpu.VMEM((chunk,), jnp.int32),
            pltpu.VMEM((chunk,), jnp.int32),
            pltpu.VMEM((chunk,), jnp.int32),
            pltpu.VMEM((chunk,), jnp.int32),
            pltpu.VMEM((chunk + LANES,), jnp.int32),
            pltpu.VMEM((chunk + 2 * LANES,), jnp.int32),
            pltpu.VMEM((chunk + 2 * LANES,), jnp.int32),
            pltpu.VMEM((npt, d), jnp.float32),
            pltpu.VMEM((npt, d), jnp.float32),
            pltpu.VMEM((grp, 2 * d), jnp.float32),
            pltpu.VMEM((grp, 2 * d), jnp.float32),
            pltpu.VMEM((8, LANES), jnp.float32),
            pltpu.SemaphoreType.DMA,
            pltpu.SemaphoreType.DMA,
            pltpu.SemaphoreType.DMA,
            pltpu.SemaphoreType.DMA,
            pltpu.SemaphoreType.DMA,
        ],
    )
    def sc_kernel(src_hbm, dst_hbm, t_hbm, xpad_hbm, m_out, part_out,
                  sbuf0, sbuf1, dbuf0, dbuf1, plist, sidx, dloc, xloc, accM,
                  gbuf0, gbuf1, stage, sem_x, sem_c0, sem_c1, sem_g0, sem_g1):
        wid = lax.axis_index("s") * 2 + lax.axis_index("c")
        base = wid * npt
        sbuf = (sbuf0, sbuf1)
        dbuf = (dbuf0, dbuf1)
        gbuf = (gbuf0, gbuf1)
        sem_c = (sem_c0, sem_c1)
        sem_g = (sem_g0, sem_g1)

        neg_inf = jnp.full((LANES,), -jnp.inf, dtype=jnp.float32)
        zerof = jnp.zeros((LANES,), jnp.float32)
        zeroi = jnp.zeros((LANES,), jnp.int32)

        xcp = pltpu.async_copy(xpad_hbm.at[pl.ds(base, npt)], xloc, sem_x)
        pltpu.async_copy(src_hbm.at[pl.ds(0, chunk)], sbuf[0], sem_c[0])
        pltpu.async_copy(dst_hbm.at[pl.ds(0, chunk)], dbuf[0], sem_c[0])

        def init_row(r, carry):
            for j in range(nb):
                accM[r, pl.ds(j * LANES, LANES)] = neg_inf
            return carry

        lax.fori_loop(0, npt, init_row, 0)

        def init_sl(i, carry):
            sidx[pl.ds(i * LANES, LANES)] = zeroi
            return carry

        lax.fori_loop(0, (chunk + 2 * LANES) // LANES, init_sl, 0)
        xcp.wait()

        z8 = tuple(jnp.zeros((LANES,), jnp.float32) for _ in range(nb))

        def chunk_pair(cp, carry):
            for b in range(2):
                ci = 2 * cp + b

                @pl.when(ci + 1 < n_chunks)
                def _():
                    nxt = pl.ds((ci + 1) * chunk, chunk)
                    pltpu.async_copy(src_hbm.at[nxt], sbuf[1 - b],
                                     sem_c[1 - b])
                    pltpu.async_copy(dst_hbm.at[nxt], dbuf[1 - b],
                                     sem_c[1 - b])

                cur = pl.ds(ci * chunk, chunk)
                pltpu.make_async_copy(src_hbm.at[cur], sbuf[b],
                                      sem_c[b]).wait()
                pltpu.make_async_copy(dst_hbm.at[cur], dbuf[b],
                                      sem_c[b]).wait()

                def filt(i, pos):
                    o = i * 2 * LANES
                    dv1 = dbuf[b][pl.ds(o, LANES)]
                    dv2 = dbuf[b][pl.ds(o + LANES, LANES)]
                    sv1 = sbuf[b][pl.ds(o, LANES)]
                    sv2 = sbuf[b][pl.ds(o + LANES, LANES)]
                    msk1 = (dv1 >= base) & (dv1 < base + npt)
                    msk2 = (dv2 >= base) & (dv2 < base + npt)
                    pc1 = plsc.all_reduce_population_count(msk1)[0]
                    pc2 = plsc.all_reduce_population_count(msk2)[0]

                    @pl.when(pc1 + pc2 > 0)
                    def _():
                        key1 = jnp.where(msk1, 0, 1).astype(jnp.int32)
                        pv1 = sv1 * 512 + ((dv1 - base) & 511)
                        _, p1 = plsc.sort_key_val(key1, pv1)
                        plist[pl.ds(pos, LANES)] = p1
                        key2 = jnp.where(msk2, 0, 1).astype(jnp.int32)
                        pv2 = sv2 * 512 + ((dv2 - base) & 511)
                        _, p2 = plsc.sort_key_val(key2, pv2)
                        plist[pl.ds(pos + pc1, LANES)] = p2

                    return pos + (pc1 + pc2)

                m_c = lax.fori_loop(0, chunk // (2 * LANES), filt,
                                    jnp.int32(0))

                def unpack(i, carry):
                    pv = plist[pl.ds(i * LANES, LANES)]
                    sidx[pl.ds(i * LANES, LANES)] = pv >> 9
                    dloc[pl.ds(i * LANES, LANES)] = pv & 511
                    return carry

                lax.fori_loop(0, (m_c + LANES - 1) // LANES, unpack, 0)
                ng = (m_c + (grp - 1)) // grp

                @pl.when(ng > 0)
                def _():
                    pltpu.async_copy(t_hbm.at[sidx.at[pl.ds(0, grp)]],
                                     gbuf[0], sem_g[0])

                def group_pair(gp, carry2):
                    for gb in range(2):
                        g = 2 * gp + gb
                        g0 = g * grp

                        @pl.when(g + 1 < ng)
                        def _():
                            pltpu.async_copy(
                                t_hbm.at[sidx.at[pl.ds(g0 + grp, grp)]],
                                gbuf[1 - gb], sem_g[1 - gb])

                        @pl.when(g < ng)
                        def _():
                            pltpu.make_async_copy(
                                t_hbm.at[sidx.at[pl.ds(g0, grp)]],
                                gbuf[gb], sem_g[gb]).wait()

                        kn = jnp.clip(m_c - g0, 0, grp)

                        def edge_body(k, carry3):
                            dacc, s2acc, crossacc = carry3
                            li = dloc[pl.ds(g0 + k, LANES)][0]
                            dn, qn, cn = [], [], []
                            for j in range(nb):
                                sl = pl.ds(j * LANES, LANES)
                                xp = gbuf[gb][k, sl]
                                xd = xloc[li, sl]
                                dn.append(dacc[j] + (xp - xd))
                                qn.append(s2acc[j] + (xp * xp + xd * xd))
                                cn.append(crossacc[j] + xp * xd)
                                vp = gbuf[gb][k, pl.ds(d + j * LANES, LANES)]
                                accM[li, sl] = jnp.maximum(accM[li, sl], vp)
                            return (tuple(dn), tuple(qn), tuple(cn))

                        carry2 = lax.fori_loop(0, kn, edge_body, carry2)
                    return carry2

                carry = lax.fori_loop(0, (ng + 1) // 2, group_pair, carry)
            return carry

        dacc, s2acc, crossacc = lax.fori_loop(0, n_chunks // 2, chunk_pair,
                                              (z8, z8, z8))

        def vsum(acc):
            t = acc[0]
            for j in range(1, nb):
                t = t + acc[j]
            return t

        stage[0, :] = vsum(dacc)
        stage[1, :] = vsum(s2acc)
        stage[2, :] = vsum(crossacc)
        for j in range(3, 8):
            stage[j, :] = zerof
        pltpu.sync_copy(stage, part_out.at[wid])
        pltpu.sync_copy(accM, m_out.at[pl.ds(base, npt)])

    return sc_kernel


def _fin_body(m_ref, u_ref, v_ref, part_ref, cvec_ref, gam_ref, bet_ref,
              pa_ref, o_ref, *, n_total):
    part = part_ref[...]
    s1 = jnp.sum(part[:, 0, :])
    sq = jnp.sum(part[:, 1, :])
    cross = jnp.sum(part[:, 2, :])
    s2 = sq - 2.0 * cross
    var = (s2 - s1 * s1 / n_total) / (n_total - 1.0)
    inv = 1.0 / (jnp.sqrt(var) + 1e-5)

    m = m_ref[...]
    agg = u_ref[...] + cvec_ref[...] + inv * (m - v_ref[...])
    agg = jnp.where(m == -jnp.inf, 0.0, agg)
    mu = jnp.mean(agg, axis=-1, keepdims=True)
    dev = agg - mu
    va = jnp.mean(dev * dev, axis=-1, keepdims=True)
    h = dev * lax.rsqrt(va + 1e-5)
    h = h * gam_ref[...] + bet_ref[...]
    o_ref[...] = jnp.where(h >= 0.0, h, pa_ref[0, 0] * h)


def _tc_fin(m, u, v, part, cvec, gam, bet, pa, n_total):
    n, d = u.shape
    rb = _row_block(n)
    nt = part.shape[0]
    return pl.pallas_call(
        functools.partial(_fin_body, n_total=float(n_total)),
        grid=(n // rb,),
        in_specs=[
            pl.BlockSpec((rb, d), lambda i: (i, 0)),
            pl.BlockSpec((rb, d), lambda i: (i, 0)),
            pl.BlockSpec((rb, d), lambda i: (i, 0)),
            pl.BlockSpec((nt, 8, LANES), lambda i: (0, 0, 0)),
            pl.BlockSpec((1, d), lambda i: (0, 0)),
            pl.BlockSpec((1, d), lambda i: (0, 0)),
            pl.BlockSpec((1, d), lambda i: (0, 0)),
            pl.BlockSpec((1, 1), lambda i: (0, 0)),
        ],
        out_specs=pl.BlockSpec((rb, d), lambda i: (i, 0)),
        out_shape=jax.ShapeDtypeStruct((n, d), jnp.float32),
    )(m, u, v, part, cvec, gam, bet, pa)


def kernel(x, edge_index, affine_w, affine_b, lin_W, lin_b, ln_gamma, ln_beta,
           prelu_a):
    n, d = x.shape
    e = edge_index.shape[1]
    src = edge_index[0].astype(jnp.int32)
    dst = edge_index[1].astype(jnp.int32)

    w1 = lin_W[:, :d]
    w2 = lin_W[:, d:]
    w1t = w1.T
    w2t = (w2 * affine_w[None, :]).T
    cvec = (affine_b @ w2.T + lin_b)[None, :]

    npt = (-(-n // N_TILES) + 7) // 8 * 8
    n_pad = N_TILES * npt
    chunk, grp = 2048, 32
    e_pad = -(-e // (2 * chunk)) * (2 * chunk)
    if e_pad != e:
        src = jnp.pad(src, (0, e_pad - e))
        dst = jnp.pad(dst, (0, e_pad - e), constant_values=jnp.int32(2 ** 30))
    xpad = jnp.pad(x, ((0, n_pad - n), (0, 0)))

    t_tab, u = _tc_pre(x, w1t, w2t)
    m_full, part = _make_sc(e_pad, n_pad, d, npt, chunk, grp)(
        src, dst, t_tab, xpad)
    out = _tc_fin(m_full[:n], u, t_tab[:, d:], part, cvec,
                  ln_gamma[None, :], ln_beta[None, :],
                  jnp.reshape(prelu_a, (1, 1)), e * d)
    return out
